# Initial kernel scaffold; baseline (speedup 1.0000x reference)
import jax
import jax.numpy as jnp
from jax.experimental import pallas as pl


def kernel(edge_attr, W0, W1, W2):
    raise NotImplementedError("write your pallas kernel here")


# speedup vs baseline: 30.8802x; 30.8802x over previous
import jax
import jax.numpy as jnp
from jax.experimental import pallas as pl

_N = 640000
_B = 5120
_D = 128


def _body(e0_ref, e1_ref, e2_ref, w0_ref, w1_ref, w2_ref, out_ref):
    acc = None
    for e_ref, w_ref, v in ((e0_ref, w0_ref, 5), (e1_ref, w1_ref, 6), (e2_ref, w2_ref, 2)):
        idx = e_ref[0]
        oh = (jax.lax.broadcasted_iota(jnp.int32, (v, _B), 0) == idx).astype(jnp.float32)
        part = jax.lax.dot_general(
            oh, w_ref[...],
            dimension_numbers=(((0,), (0,)), ((), ())),
            preferred_element_type=jnp.float32,
        )
        acc = part if acc is None else acc + part
    out_ref[...] = acc


def kernel(edge_attr, W0, W1, W2):
    nb = _N // _B
    ea = edge_attr.astype(jnp.int32)
    e0 = ea[:, 0].reshape(nb, 1, _B)
    e1 = ea[:, 1].reshape(nb, 1, _B)
    e2 = ea[:, 2].reshape(nb, 1, _B)
    espec = pl.BlockSpec((1, 1, _B), lambda i: (i, 0, 0))
    wspec = lambda v: pl.BlockSpec((v, _D), lambda i: (0, 0))
    return pl.pallas_call(
        _body,
        grid=(nb,),
        in_specs=[espec, espec, espec, wspec(5), wspec(6), wspec(2)],
        out_specs=pl.BlockSpec((_B, _D), lambda i: (i, 0)),
        out_shape=jax.ShapeDtypeStruct((_N, _D), jnp.float32),
    )(e0, e1, e2, W0, W1, W2)
